# Initial kernel scaffold; baseline (speedup 1.0000x reference)
import jax
import jax.numpy as jnp
from jax.experimental import pallas as pl


def kernel(h, edge_index, r, norm, emb_table, basis, w_comp, loop_weight, h_bias):
    raise NotImplementedError("write your pallas kernel here")


# speedup vs baseline: 2.2073x; 2.2073x over previous
import functools

import jax
import jax.numpy as jnp
from jax import lax
from jax.experimental import pallas as pl
from jax.experimental.pallas import tpu as pltpu
from jax.experimental.pallas import tpu_sc as plsc

N_NODES = 10000
H = 100
E = 320000
NB = 4
NREL = 474

CHUNK = 80
TILES = 16
EDGES_PER_TILE = E // TILES
NCHUNK = EDGES_PER_TILE // CHUNK
HALF = N_NODES // 2
HP = 112

ZR_A = 312
ZR_LAST = HALF - (TILES - 1) * ZR_A
ZBUF = 104

_WINDOWS = tuple(range(0, HP, 16))


def _sc_accumulate(x, src, dst, rel, norm_flat, w_flat):
    mesh = plsc.VectorSubcoreMesh(core_axis_name="c", subcore_axis_name="s")

    @functools.partial(
        pl.kernel,
        mesh=mesh,
        out_type=jax.ShapeDtypeStruct((NB, N_NODES, HP), jnp.float32),
        compiler_params=pltpu.CompilerParams(
            needs_layout_passes=False, use_tc_tiling_on_sc=False),
        scratch_types=[
            pltpu.VMEM_SHARED((HALF, HP), jnp.float32),
            pltpu.VMEM_SHARED((HALF, HP), jnp.float32),
            pltpu.VMEM((NREL * NB,), jnp.float32),
            pltpu.VMEM((CHUNK,), jnp.int32),
            pltpu.VMEM((CHUNK,), jnp.int32),
            pltpu.VMEM((CHUNK,), jnp.int32),
            pltpu.VMEM((CHUNK,), jnp.int32),
            pltpu.VMEM((CHUNK,), jnp.float32),
            pltpu.VMEM((CHUNK,), jnp.float32),
            pltpu.VMEM((CHUNK,), jnp.float32),
            pltpu.VMEM((CHUNK, HP), jnp.float32),
            pltpu.VMEM((CHUNK, HP), jnp.float32),
            pltpu.VMEM((CHUNK, HP), jnp.float32),
            pltpu.VMEM((ZBUF, HP), jnp.float32),
            pltpu.SemaphoreType.DMA,
        ],
    )
    def k(x_hbm, src_hbm, dst_hbm, r_hbm, norm_hbm, w_hbm, out_hbm,
          acc0, acc1, w_v, src_v, dst_v, rel_v, cdst_v, norm_v, c0_v, c1_v,
          rows_v, s0_v, s1_v, z_v, sem):
        c = lax.axis_index("c")
        s = lax.axis_index("s")

        pltpu.sync_copy(w_hbm, w_v)

        zv = jnp.zeros((16,), jnp.float32)

        def zrow(i, carry):
            for off in _WINDOWS:
                z_v[i, pl.ds(off, 16)] = zv
            return carry

        lax.fori_loop(0, ZBUF, zrow, 0)

        b0 = c * 2
        rr = s * ZR_A

        for p in range(2):
            lo = p * HALF

            for blk in range(ZR_A // ZBUF):
                pltpu.sync_copy(z_v, acc0.at[pl.ds(rr + blk * ZBUF, ZBUF)])
                pltpu.sync_copy(z_v, acc1.at[pl.ds(rr + blk * ZBUF, ZBUF)])

            @pl.when(s == TILES - 1)
            def _():
                extra = ZR_LAST - ZR_A
                pltpu.sync_copy(z_v.at[pl.ds(0, extra)],
                                acc0.at[pl.ds(rr + ZR_A, extra)])
                pltpu.sync_copy(z_v.at[pl.ds(0, extra)],
                                acc1.at[pl.ds(rr + ZR_A, extra)])

            plsc.subcore_barrier()

            def chunk_body(i, carry):
                base = s * EDGES_PER_TILE + i * CHUNK
                pltpu.sync_copy(src_hbm.at[pl.ds(base, CHUNK)], src_v)
                pltpu.sync_copy(dst_hbm.at[pl.ds(base, CHUNK)], dst_v)
                pltpu.sync_copy(r_hbm.at[pl.ds(base, CHUNK)], rel_v)
                pltpu.sync_copy(norm_hbm.at[pl.ds(base, CHUNK)], norm_v)
                pltpu.async_copy(x_hbm.at[src_v], rows_v, sem).wait()

                zero16 = jnp.zeros((16,), jnp.float32)

                def coeffs(j, carry2):
                    sl = pl.ds(j * 16, 16)
                    rv = rel_v[sl]
                    nv = norm_v[sl]
                    dv = dst_v[sl]
                    inr = (dv >= lo) & (dv < lo + HALF)
                    i0 = rv * NB + b0
                    c0 = plsc.load_gather(w_v, [i0]) * nv
                    c1 = plsc.load_gather(w_v, [i0 + 1]) * nv
                    c0_v[sl] = jnp.where(inr, c0, zero16)
                    c1_v[sl] = jnp.where(inr, c1, zero16)
                    cdst_v[sl] = jnp.where(inr, dv - lo, 0)
                    return carry2

                lax.fori_loop(0, CHUNK // 16, coeffs, 0)

                def egroup(g, carry2):
                    c0g = c0_v[pl.ds(g * 16, 16)]
                    c1g = c1_v[pl.ds(g * 16, 16)]
                    for j in range(16):
                        e = g * 16 + j
                        f0 = c0g[j]
                        f1 = c1g[j]
                        for off in _WINDOWS:
                            v = rows_v[e, pl.ds(off, 16)]
                            s0_v[e, pl.ds(off, 16)] = v * f0
                            s1_v[e, pl.ds(off, 16)] = v * f1
                    return carry2

                lax.fori_loop(0, CHUNK // 16, egroup, 0)

                pltpu.sync_copy(s0_v, acc0.at[cdst_v], add=True)
                pltpu.sync_copy(s1_v, acc1.at[cdst_v], add=True)
                return carry

            lax.fori_loop(0, NCHUNK, chunk_body, 0)
            plsc.subcore_barrier()

            @pl.when(s < TILES - 1)
            def _():
                pltpu.sync_copy(acc0.at[pl.ds(rr, ZR_A)],
                                out_hbm.at[b0, pl.ds(lo + rr, ZR_A)])
                pltpu.sync_copy(acc1.at[pl.ds(rr, ZR_A)],
                                out_hbm.at[b0 + 1, pl.ds(lo + rr, ZR_A)])

            @pl.when(s == TILES - 1)
            def _():
                pltpu.sync_copy(acc0.at[pl.ds(rr, ZR_LAST)],
                                out_hbm.at[b0, pl.ds(lo + rr, ZR_LAST)])
                pltpu.sync_copy(acc1.at[pl.ds(rr, ZR_LAST)],
                                out_hbm.at[b0 + 1, pl.ds(lo + rr, ZR_LAST)])

    return k(x, src, dst, rel, norm_flat, w_flat)


def _tc_combine(acc, x, basis_pad, loop_weight, h_bias2d):
    BLK = 2000

    def body(acc_ref, x_ref, b_ref, lw_ref, bias_ref, o_ref):
        out = jnp.dot(x_ref[...], lw_ref[...], preferred_element_type=jnp.float32)
        for b in range(NB):
            out = out + jnp.dot(acc_ref[b], b_ref[b],
                                preferred_element_type=jnp.float32)
        o_ref[...] = out + bias_ref[...]

    return pl.pallas_call(
        body,
        grid=(N_NODES // BLK,),
        in_specs=[
            pl.BlockSpec((NB, BLK, HP), lambda i: (0, i, 0)),
            pl.BlockSpec((BLK, H), lambda i: (i, 0)),
            pl.BlockSpec((NB, HP, H), lambda i: (0, 0, 0)),
            pl.BlockSpec((H, H), lambda i: (0, 0)),
            pl.BlockSpec((1, H), lambda i: (0, 0)),
        ],
        out_specs=pl.BlockSpec((BLK, H), lambda i: (i, 0)),
        out_shape=jax.ShapeDtypeStruct((N_NODES, H), jnp.float32),
    )(acc, x, basis_pad, loop_weight, h_bias2d)


def kernel(h, edge_index, r, norm, emb_table, basis, w_comp, loop_weight, h_bias):
    x = jnp.take(emb_table, h, axis=0)
    x_pad = jnp.pad(x, ((0, 0), (0, HP - H)))
    basis_pad = jnp.pad(basis, ((0, 0), (0, HP - H), (0, 0)))
    acc = _sc_accumulate(x_pad, edge_index[0], edge_index[1], r,
                         norm.reshape(-1), w_comp.reshape(-1))
    return _tc_combine(acc, x, basis_pad, loop_weight, h_bias.reshape(1, H))


# speedup vs baseline: 8.4052x; 3.8079x over previous
import functools

import jax
import jax.numpy as jnp
from jax import lax
from jax.experimental import pallas as pl
from jax.experimental.pallas import tpu as pltpu
from jax.experimental.pallas import tpu_sc as plsc

N_NODES = 10000
H = 100
E = 320000
NB = 4
NREL = 474

DW = 64
HSPLIT = 56
CHUNK = 80
TILES = 16
EDGES_PER_TILE = E // TILES
NCHUNK = EDGES_PER_TILE // CHUNK

ZR_A = 632
ZR_LAST = N_NODES - (TILES - 1) * ZR_A
ZBUF = 104

_WINDOWS = tuple(range(0, DW, 16))


def _sc_accumulate(x0, x1, src, dst, rel, norm_flat, w_flat):
    mesh = plsc.VectorSubcoreMesh(core_axis_name="c", subcore_axis_name="s")

    @functools.partial(
        pl.kernel,
        mesh=mesh,
        out_type=jax.ShapeDtypeStruct((NB, 2, N_NODES, DW), jnp.float32),
        compiler_params=pltpu.CompilerParams(
            needs_layout_passes=False, use_tc_tiling_on_sc=False),
        scratch_types=[
            pltpu.VMEM_SHARED((N_NODES, DW), jnp.float32),
            pltpu.VMEM_SHARED((N_NODES, DW), jnp.float32),
            pltpu.VMEM((NREL * NB,), jnp.float32),
            pltpu.VMEM((2, CHUNK), jnp.int32),
            pltpu.VMEM((2, CHUNK), jnp.int32),
            pltpu.VMEM((2, CHUNK), jnp.int32),
            pltpu.VMEM((2, CHUNK), jnp.float32),
            pltpu.VMEM((2, CHUNK), jnp.int32),
            pltpu.VMEM((2, CHUNK), jnp.float32),
            pltpu.VMEM((2, CHUNK), jnp.float32),
            pltpu.VMEM((2, CHUNK, DW), jnp.float32),
            pltpu.VMEM((2, CHUNK, DW), jnp.float32),
            pltpu.VMEM((2, CHUNK, DW), jnp.float32),
            pltpu.VMEM((ZBUF, DW), jnp.float32),
            (pltpu.SemaphoreType.DMA, pltpu.SemaphoreType.DMA),
            (pltpu.SemaphoreType.DMA, pltpu.SemaphoreType.DMA),
            (pltpu.SemaphoreType.DMA, pltpu.SemaphoreType.DMA),
        ],
    )
    def k(x0_hbm, x1_hbm, src_hbm, dst_hbm, r_hbm, norm_hbm, w_hbm, out_hbm,
          acc0, acc1, w_v, srcb, dstb, relb, normb, sdst, c0b, c1b,
          rowsb, s0b, s1b, z_v, msem, gsem, ssem):
        c = lax.axis_index("c")
        s = lax.axis_index("s")

        pltpu.sync_copy(w_hbm, w_v)

        zv = jnp.zeros((16,), jnp.float32)

        def zrow(i, carry):
            for off in _WINDOWS:
                z_v[i, pl.ds(off, 16)] = zv
            return carry

        lax.fori_loop(0, ZBUF, zrow, 0)

        b0 = c * 2
        rr = s * ZR_A
        ebase = s * EDGES_PER_TILE

        def issue_meta(i, par):
            base = ebase + i * CHUNK
            pltpu.async_copy(src_hbm.at[pl.ds(base, CHUNK)], srcb.at[par], msem[par])
            pltpu.async_copy(dst_hbm.at[pl.ds(base, CHUNK)], dstb.at[par], msem[par])
            pltpu.async_copy(r_hbm.at[pl.ds(base, CHUNK)], relb.at[par], msem[par])
            pltpu.async_copy(norm_hbm.at[pl.ds(base, CHUNK)], normb.at[par], msem[par])

        def wait_meta(par):
            pltpu.make_async_copy(src_hbm.at[pl.ds(0, CHUNK)], srcb.at[par], msem[par]).wait()
            pltpu.make_async_copy(dst_hbm.at[pl.ds(0, CHUNK)], dstb.at[par], msem[par]).wait()
            pltpu.make_async_copy(r_hbm.at[pl.ds(0, CHUNK)], relb.at[par], msem[par]).wait()
            pltpu.make_async_copy(norm_hbm.at[pl.ds(0, CHUNK)], normb.at[par],
                                  msem[par]).wait()

        for p in range(2):
            x_hbm = x0_hbm if p == 0 else x1_hbm

            for blk in range(5):
                pltpu.sync_copy(z_v, acc0.at[pl.ds(rr + blk * ZBUF, ZBUF)])
                pltpu.sync_copy(z_v, acc1.at[pl.ds(rr + blk * ZBUF, ZBUF)])

            @pl.when(s < TILES - 1)
            def _():
                pltpu.sync_copy(z_v, acc0.at[pl.ds(rr + 5 * ZBUF, ZBUF)])
                pltpu.sync_copy(z_v, acc1.at[pl.ds(rr + 5 * ZBUF, ZBUF)])
                pltpu.sync_copy(z_v.at[pl.ds(0, 8)], acc0.at[pl.ds(rr + 624, 8)])
                pltpu.sync_copy(z_v.at[pl.ds(0, 8)], acc1.at[pl.ds(rr + 624, 8)])

            plsc.subcore_barrier()

            issue_meta(0, 0)
            issue_meta(1, 1)
            wait_meta(0)
            pltpu.async_copy(x_hbm.at[srcb.at[0]], rowsb.at[0], gsem[0])

            def scatter_wait(par):
                pltpu.make_async_copy(s0b.at[par], acc0.at[sdst.at[par]], ssem[par]).wait()
                pltpu.make_async_copy(s1b.at[par], acc1.at[sdst.at[par]], ssem[par]).wait()

            def step(kk, i, par):
                @pl.when(i + 1 < NCHUNK)
                def _():
                    wait_meta(1 - par)
                    pltpu.async_copy(x_hbm.at[srcb.at[1 - par]], rowsb.at[1 - par],
                                     gsem[1 - par])

                @pl.when(kk >= 1)
                def _():
                    scatter_wait(par)

                def coeffs(j, carry2):
                    sl = pl.ds(j * 16, 16)
                    rv = relb[par, sl]
                    nv = normb[par, sl]
                    i0 = rv * NB + b0
                    c0b[par, sl] = plsc.load_gather(w_v, [i0]) * nv
                    c1b[par, sl] = plsc.load_gather(w_v, [i0 + 1]) * nv
                    sdst[par, sl] = dstb[par, sl]
                    return carry2

                lax.fori_loop(0, CHUNK // 16, coeffs, 0)

                pltpu.make_async_copy(x_hbm.at[srcb.at[par]], rowsb.at[par],
                                      gsem[par]).wait()

                def egroup(g, carry2):
                    c0g = c0b[par, pl.ds(g * 16, 16)]
                    c1g = c1b[par, pl.ds(g * 16, 16)]
                    for j in range(16):
                        e = g * 16 + j
                        f0 = c0g[j]
                        f1 = c1g[j]
                        for off in _WINDOWS:
                            v = rowsb[par, e, pl.ds(off, 16)]
                            s0b[par, e, pl.ds(off, 16)] = v * f0
                            s1b[par, e, pl.ds(off, 16)] = v * f1
                    return carry2

                lax.fori_loop(0, CHUNK // 16, egroup, 0)

                pltpu.async_copy(s0b.at[par], acc0.at[sdst.at[par]], ssem[par],
                                 add=True)
                pltpu.async_copy(s1b.at[par], acc1.at[sdst.at[par]], ssem[par],
                                 add=True)

                @pl.when(i + 2 < NCHUNK)
                def _():
                    issue_meta(i + 2, par)

            def pipe(kk, carry):
                step(kk, 2 * kk, 0)
                step(kk, 2 * kk + 1, 1)
                return carry

            lax.fori_loop(0, NCHUNK // 2, pipe, 0)
            scatter_wait(0)
            scatter_wait(1)
            plsc.subcore_barrier()

            @pl.when(s < TILES - 1)
            def _():
                pltpu.sync_copy(acc0.at[pl.ds(rr, ZR_A)],
                                out_hbm.at[b0, p, pl.ds(rr, ZR_A)])
                pltpu.sync_copy(acc1.at[pl.ds(rr, ZR_A)],
                                out_hbm.at[b0 + 1, p, pl.ds(rr, ZR_A)])

            @pl.when(s == TILES - 1)
            def _():
                pltpu.sync_copy(acc0.at[pl.ds(rr, ZR_LAST)],
                                out_hbm.at[b0, p, pl.ds(rr, ZR_LAST)])
                pltpu.sync_copy(acc1.at[pl.ds(rr, ZR_LAST)],
                                out_hbm.at[b0 + 1, p, pl.ds(rr, ZR_LAST)])

    return k(x0, x1, src, dst, rel, norm_flat, w_flat)


def _tc_combine(acc, x, basis_split, loop_weight, h_bias2d):
    BLK = 2000

    def body(acc_ref, x_ref, b_ref, lw_ref, bias_ref, o_ref):
        out = jnp.dot(x_ref[...], lw_ref[...], preferred_element_type=jnp.float32)
        for b in range(NB):
            for hh in range(2):
                out = out + jnp.dot(acc_ref[b, hh], b_ref[b, hh],
                                    preferred_element_type=jnp.float32)
        o_ref[...] = out + bias_ref[...]

    return pl.pallas_call(
        body,
        grid=(N_NODES // BLK,),
        in_specs=[
            pl.BlockSpec((NB, 2, BLK, DW), lambda i: (0, 0, i, 0)),
            pl.BlockSpec((BLK, H), lambda i: (i, 0)),
            pl.BlockSpec((NB, 2, DW, H), lambda i: (0, 0, 0, 0)),
            pl.BlockSpec((H, H), lambda i: (0, 0)),
            pl.BlockSpec((1, H), lambda i: (0, 0)),
        ],
        out_specs=pl.BlockSpec((BLK, H), lambda i: (i, 0)),
        out_shape=jax.ShapeDtypeStruct((N_NODES, H), jnp.float32),
    )(acc, x, basis_split, loop_weight, h_bias2d)


def kernel(h, edge_index, r, norm, emb_table, basis, w_comp, loop_weight, h_bias):
    x = jnp.take(emb_table, h, axis=0)
    x0 = jnp.pad(x[:, :HSPLIT], ((0, 0), (0, DW - HSPLIT)))
    x1 = jnp.pad(x[:, HSPLIT:], ((0, 0), (0, DW - (H - HSPLIT))))
    bs0 = jnp.pad(basis[:, :HSPLIT, :], ((0, 0), (0, DW - HSPLIT), (0, 0)))
    bs1 = jnp.pad(basis[:, HSPLIT:, :], ((0, 0), (0, DW - (H - HSPLIT)), (0, 0)))
    basis_split = jnp.stack([bs0, bs1], axis=1)
    acc = _sc_accumulate(x0, x1, edge_index[0], edge_index[1], r,
                         norm.reshape(-1), w_comp.reshape(-1))
    return _tc_combine(acc, x, basis_split, loop_weight, h_bias.reshape(1, H))
